# Initial kernel scaffold; baseline (speedup 1.0000x reference)
import jax
import jax.numpy as jnp
from jax.experimental import pallas as pl


def kernel(x, pre_norm_w, post_norm_w, Wq, Wk, Wv, Wo, Wg, W1, W2):
    raise NotImplementedError("write your pallas kernel here")


# speedup vs baseline: 1.2477x; 1.2477x over previous
import jax
import jax.numpy as jnp
from jax.experimental import pallas as pl

B, S, D, H, E, K, HID = 1, 2048, 1024, 16, 8, 2, 1024
DH = D // H
EPS = 1e-05
EPAD = 128

BS_QKV = 512
BQ = 256
BS_POST = 512
BS_MOE = 1024


def _qkv_body(x_ref, w_ref, wqkv_ref, o_ref):
    x = x_ref[...]
    ms = jnp.mean(x * x, axis=-1, keepdims=True)
    xn = x * jax.lax.rsqrt(ms + EPS) * w_ref[...]
    o_ref[...] = jnp.dot(
        xn.astype(jnp.bfloat16), wqkv_ref[...],
        preferred_element_type=jnp.float32).astype(jnp.bfloat16)


def _attn_body(q_ref, k_ref, v_ref, o_ref):
    qi = pl.program_id(1)
    q = q_ref[0, 0]
    k = k_ref[0, 0]
    v = v_ref[0, 0]
    s = jax.lax.dot_general(
        q, k, (((1,), (1,)), ((), ())),
        preferred_element_type=jnp.float32) * (1.0 / (DH ** 0.5))
    rows = qi * BQ + jax.lax.broadcasted_iota(jnp.int32, (BQ, S), 0)
    cols = jax.lax.broadcasted_iota(jnp.int32, (BQ, S), 1)
    s = jnp.where(cols <= rows, s, -1e30)
    m = jnp.max(s, axis=-1, keepdims=True)
    p = jnp.exp(s - m)
    l = jnp.sum(p, axis=-1, keepdims=True)
    p = (p / l).astype(jnp.bfloat16)
    o_ref[0] = jnp.dot(p, v, preferred_element_type=jnp.float32).astype(jnp.bfloat16)


def _post_body(a_ref, wo_ref, x_ref, pw_ref, wg_ref, psum_ref, pn_ref, lg_ref):
    o = jnp.dot(a_ref[...], wo_ref[...], preferred_element_type=jnp.float32)
    ps = o + x_ref[...]
    psum_ref[...] = ps
    ms = jnp.mean(ps * ps, axis=-1, keepdims=True)
    pn = ps * jax.lax.rsqrt(ms + EPS) * pw_ref[...]
    pn_ref[...] = pn.astype(jnp.bfloat16)
    lg_ref[...] = jnp.dot(pn, wg_ref[...], preferred_element_type=jnp.float32)


def _router_body(lg_ref, comb_ref, aux_ref):
    lg = lg_ref[...]
    lanes = jax.lax.broadcasted_iota(jnp.int32, (S, EPAD), 1)
    valid = lanes < E
    l = jnp.where(valid, lg, -1e30)
    m = jnp.max(l, axis=1, keepdims=True)
    ex = jnp.where(valid, jnp.exp(l - m), 0.0)
    probs = ex / jnp.sum(ex, axis=1, keepdims=True)
    v1 = jnp.max(probs, axis=1, keepdims=True)
    i1 = jnp.min(jnp.where(probs == v1, lanes, EPAD), axis=1, keepdims=True)
    mask1 = lanes == i1
    probs2 = jnp.where(mask1, -1.0, probs)
    v2 = jnp.max(probs2, axis=1, keepdims=True)
    i2 = jnp.min(jnp.where(probs2 == v2, lanes, EPAD), axis=1, keepdims=True)
    tot = v1 + v2
    comb = jnp.where(mask1, v1 / tot, jnp.where(lanes == i2, v2 / tot, 0.0))
    comb_ref[...] = comb
    sel = (mask1 | (lanes == i2)).astype(jnp.float32)
    frac = jnp.mean(sel, axis=0, keepdims=True)
    pmean = jnp.mean(probs, axis=0, keepdims=True)
    aux_ref[...] = (E / K) * jnp.sum(frac * pmean, keepdims=True)


def _moe_body(pn_ref, c_ref, psum_ref, w1_ref, w2_ref, o_ref):
    e = pl.program_id(1)
    x = pn_ref[...]
    h = jnp.dot(x, w1_ref[0], preferred_element_type=jnp.float32)
    h = jnp.maximum(h, 0.0).astype(jnp.bfloat16)
    part = jnp.dot(h, w2_ref[0], preferred_element_type=jnp.float32)
    lanes = jax.lax.broadcasted_iota(jnp.int32, (BS_MOE, EPAD), 1)
    w = jnp.sum(jnp.where(lanes == e, c_ref[...], 0.0), axis=1, keepdims=True)
    contrib = w * part

    @pl.when(e == 0)
    def _():
        o_ref[...] = psum_ref[...] + contrib

    @pl.when(e > 0)
    def _():
        o_ref[...] += contrib


def kernel(x, pre_norm_w, post_norm_w, Wq, Wk, Wv, Wo, Wg, W1, W2):
    xf = x.reshape(S, D)
    wqkv = jnp.concatenate([Wq, Wk, Wv], axis=1).astype(jnp.bfloat16)

    qkv = pl.pallas_call(
        _qkv_body,
        grid=(S // BS_QKV,),
        in_specs=[
            pl.BlockSpec((BS_QKV, D), lambda i: (i, 0)),
            pl.BlockSpec((1, D), lambda i: (0, 0)),
            pl.BlockSpec((D, 3 * D), lambda i: (0, 0)),
        ],
        out_specs=pl.BlockSpec((BS_QKV, 3 * D), lambda i: (i, 0)),
        out_shape=jax.ShapeDtypeStruct((S, 3 * D), jnp.bfloat16),
    )(xf, pre_norm_w.reshape(1, D), wqkv)

    qkv_t = qkv.reshape(S, 3, H, DH).transpose(1, 2, 0, 3)

    attn_h = pl.pallas_call(
        _attn_body,
        grid=(H, S // BQ),
        in_specs=[
            pl.BlockSpec((1, 1, BQ, DH), lambda h, qi: (0, h, qi, 0)),
            pl.BlockSpec((1, 1, S, DH), lambda h, qi: (1, h, 0, 0)),
            pl.BlockSpec((1, 1, S, DH), lambda h, qi: (2, h, 0, 0)),
        ],
        out_specs=pl.BlockSpec((1, BQ, DH), lambda h, qi: (h, qi, 0)),
        out_shape=jax.ShapeDtypeStruct((H, S, DH), jnp.bfloat16),
    )(qkv_t, qkv_t, qkv_t)
    attn = attn_h.transpose(1, 0, 2).reshape(S, D)

    wg_pad = jnp.pad(Wg, ((0, 0), (0, EPAD - E)))
    post_sum, pn16, logits = pl.pallas_call(
        _post_body,
        grid=(S // BS_POST,),
        in_specs=[
            pl.BlockSpec((BS_POST, D), lambda i: (i, 0)),
            pl.BlockSpec((D, D), lambda i: (0, 0)),
            pl.BlockSpec((BS_POST, D), lambda i: (i, 0)),
            pl.BlockSpec((1, D), lambda i: (0, 0)),
            pl.BlockSpec((D, EPAD), lambda i: (0, 0)),
        ],
        out_specs=[
            pl.BlockSpec((BS_POST, D), lambda i: (i, 0)),
            pl.BlockSpec((BS_POST, D), lambda i: (i, 0)),
            pl.BlockSpec((BS_POST, EPAD), lambda i: (i, 0)),
        ],
        out_shape=[
            jax.ShapeDtypeStruct((S, D), jnp.float32),
            jax.ShapeDtypeStruct((S, D), jnp.bfloat16),
            jax.ShapeDtypeStruct((S, EPAD), jnp.float32),
        ],
    )(attn, Wo.astype(jnp.bfloat16), xf, post_norm_w.reshape(1, D), wg_pad)

    comb, aux = pl.pallas_call(
        _router_body,
        in_specs=[pl.BlockSpec((S, EPAD), lambda: (0, 0))],
        out_specs=[
            pl.BlockSpec((S, EPAD), lambda: (0, 0)),
            pl.BlockSpec((1, 1), lambda: (0, 0)),
        ],
        out_shape=[
            jax.ShapeDtypeStruct((S, EPAD), jnp.float32),
            jax.ShapeDtypeStruct((1, 1), jnp.float32),
        ],
    )(logits)

    out = pl.pallas_call(
        _moe_body,
        grid=(S // BS_MOE, E),
        in_specs=[
            pl.BlockSpec((BS_MOE, D), lambda i, e: (i, 0)),
            pl.BlockSpec((BS_MOE, EPAD), lambda i, e: (i, 0)),
            pl.BlockSpec((BS_MOE, D), lambda i, e: (i, 0)),
            pl.BlockSpec((1, D, HID), lambda i, e: (e, 0, 0)),
            pl.BlockSpec((1, HID, D), lambda i, e: (e, 0, 0)),
        ],
        out_specs=pl.BlockSpec((BS_MOE, D), lambda i, e: (i, 0)),
        out_shape=jax.ShapeDtypeStruct((S, D), jnp.float32),
    )(pn16, comb, post_sum, W1.astype(jnp.bfloat16), W2.astype(jnp.bfloat16))

    return out.reshape(B, S, D), aux.reshape(())
